# Initial kernel scaffold; baseline (speedup 1.0000x reference)
import jax
import jax.numpy as jnp
from jax.experimental import pallas as pl


def kernel(x, edge_index, edge_attr, batch, W_nn, b_nn, W_edge, b_edge, bn_gamma, bn_beta, Wr1, br1, Wr2, br2, Wr3, br3):
    raise NotImplementedError("write your pallas kernel here")


# speedup vs baseline: 2.0127x; 2.0127x over previous
import functools

import jax
import jax.numpy as jnp
from jax import lax
from jax.experimental import pallas as pl
from jax.experimental.pallas import tpu as pltpu
from jax.experimental.pallas import tpu_sc as plsc

_NC = 2
_NS = 16
_LANES = 16


@functools.lru_cache(maxsize=None)
def _make_edge_kernel(n, e_edges, d):
    nw = _NC * _NS
    epw = e_edges // nw
    ch = 80
    assert epw % ch == 0
    nchunk = epw // ch
    zr = 200
    assert n % zr == 0
    nzchunk = n // zr
    vpr = d // _LANES

    mesh = plsc.VectorSubcoreMesh(core_axis_name="c", subcore_axis_name="s")

    @functools.partial(
        pl.kernel,
        mesh=mesh,
        out_type=jax.ShapeDtypeStruct((_NC, n, d), jnp.float32),
        scratch_types=[
            pltpu.VMEM((ch,), jnp.int32),
            pltpu.VMEM((ch,), jnp.int32),
            pltpu.VMEM((ch, d), jnp.float32),
            pltpu.VMEM((ch, d), jnp.float32),
            pltpu.VMEM((zr, d), jnp.float32),
            pltpu.VMEM_SHARED((n, d), jnp.float32),
            pltpu.SemaphoreType.DMA,
        ],
    )
    def edge_kernel(h_hbm, e_hbm, src_hbm, dst_hbm, out_hbm,
                    src_v, dst_v, rows_v, erows_v, zero_v, aggr_sh, sem):
        c = lax.axis_index("c")
        s = lax.axis_index("s")

        zeros16 = jnp.zeros((_LANES,), jnp.float32)

        def zbody(i, _):
            for q in range(vpr):
                zero_v[i, pl.ds(q * _LANES, _LANES)] = zeros16
            return 0

        lax.fori_loop(0, zr, zbody, 0)
        for j in range((nzchunk + _NS - 1) // _NS):
            k = s + j * _NS

            @pl.when(k < nzchunk)
            def _():
                pltpu.sync_copy(zero_v, aggr_sh.at[pl.ds(k * zr, zr)])

        plsc.subcore_barrier()

        base = (c * _NS + s) * epw

        def body(i, _):
            off = base + i * ch
            pltpu.sync_copy(src_hbm.at[pl.ds(off, ch)], src_v)
            pltpu.sync_copy(dst_hbm.at[pl.ds(off, ch)], dst_v)
            pltpu.sync_copy(e_hbm.at[pl.ds(off, ch)], erows_v)
            pltpu.async_copy(h_hbm.at[src_v], rows_v, sem).wait()

            def cbody(r, _):
                for q in range(vpr):
                    v = (rows_v[r, pl.ds(q * _LANES, _LANES)]
                         + erows_v[r, pl.ds(q * _LANES, _LANES)])
                    rows_v[r, pl.ds(q * _LANES, _LANES)] = jnp.maximum(v, 0.0)
                return 0

            lax.fori_loop(0, ch, cbody, 0)
            pltpu.sync_copy(rows_v, aggr_sh.at[dst_v], add=True)
            return 0

        lax.fori_loop(0, nchunk, body, 0)
        plsc.subcore_barrier()

        @pl.when(s == 0)
        def _():
            pltpu.sync_copy(aggr_sh, out_hbm.at[c])

    return edge_kernel


@functools.lru_cache(maxsize=None)
def _make_eproj(num_layers, e_edges, ed, d):
    eb = 3200
    assert e_edges % eb == 0
    grid = (num_layers, e_edges // eb)

    def body(ea_ref, we_ref, be_ref, out_ref):
        out_ref[0] = (jnp.dot(ea_ref[...], we_ref[0],
                              preferred_element_type=jnp.float32)
                      + be_ref[0])

    return pl.pallas_call(
        body,
        grid=grid,
        in_specs=[
            pl.BlockSpec((eb, ed), lambda l, i: (i, 0)),
            pl.BlockSpec((1, ed, d), lambda l, i: (l, 0, 0)),
            pl.BlockSpec((1, 1, d), lambda l, i: (l, 0, 0)),
        ],
        out_specs=pl.BlockSpec((1, eb, d), lambda l, i: (l, i, 0)),
        out_shape=jax.ShapeDtypeStruct((num_layers, e_edges, d), jnp.float32),
    )


@functools.lru_cache(maxsize=None)
def _make_update(n, d, h_dim):
    def body(h_ref, a_ref, w_ref, b_ref, g_ref, bt_ref, out_ref):
        t = h_ref[...] + a_ref[0] + a_ref[1]
        h2 = jnp.dot(t, w_ref[...], preferred_element_type=jnp.float32) + b_ref[...]
        mean = jnp.mean(h2, axis=0, keepdims=True)
        dvt = h2 - mean
        var = jnp.mean(dvt * dvt, axis=0, keepdims=True)
        out_ref[...] = jnp.maximum(
            g_ref[...] * dvt * lax.rsqrt(var + 1e-5) + bt_ref[...], 0.0)

    return pl.pallas_call(
        body,
        out_shape=jax.ShapeDtypeStruct((n, h_dim), jnp.float32),
    )


@functools.lru_cache(maxsize=None)
def _make_readout(n, h_dim, g):
    def body(h_ref, seg_ref, w1_ref, b1_ref, w2_ref, b2_ref, w3_ref, b3_ref,
             out_ref):
        onehot = jnp.where(
            seg_ref[...] == lax.broadcasted_iota(jnp.int32, (1, g), 1),
            1.0, 0.0)
        dn = (((0,), (0,)), ((), ()))
        pooled = lax.dot_general(onehot, h_ref[...], dn,
                                 preferred_element_type=jnp.float32)
        counts = lax.dot_general(onehot, jnp.ones((n, 1), jnp.float32), dn,
                                 preferred_element_type=jnp.float32)
        pooled = pooled / jnp.maximum(counts, 1.0)
        z = jnp.maximum(jnp.dot(pooled, w1_ref[...],
                                preferred_element_type=jnp.float32)
                        + b1_ref[...], 0.0)
        z = jnp.maximum(jnp.dot(z, w2_ref[...],
                                preferred_element_type=jnp.float32)
                        + b2_ref[...], 0.0)
        out_ref[...] = (jnp.dot(z, w3_ref[...],
                                preferred_element_type=jnp.float32)
                        + b3_ref[...])

    return pl.pallas_call(
        body,
        out_shape=jax.ShapeDtypeStruct((g, 1), jnp.float32),
    )


def kernel(x, edge_index, edge_attr, batch, W_nn, b_nn, W_edge, b_edge,
           bn_gamma, bn_beta, Wr1, br1, Wr2, br2, Wr3, br3):
    n, d = x.shape
    num_layers, ed, h_dim = W_edge.shape
    e_edges = edge_attr.shape[0]
    g = 64

    src = edge_index[0]
    dst = edge_index[1]

    eproj = _make_eproj(num_layers, e_edges, ed, d)(
        edge_attr, W_edge, b_edge.reshape(num_layers, 1, h_dim))

    edge_call = _make_edge_kernel(n, e_edges, d)
    upd_call = _make_update(n, d, h_dim)

    h = x
    for l in range(num_layers):
        aggr2 = edge_call(h, eproj[l], src, dst)
        h = upd_call(h, aggr2, W_nn[l],
                     b_nn[l].reshape(1, h_dim),
                     bn_gamma[l].reshape(1, h_dim),
                     bn_beta[l].reshape(1, h_dim))

    out = _make_readout(n, h_dim, g)(
        h, batch.reshape(n, 1),
        Wr1, br1.reshape(1, -1), Wr2, br2.reshape(1, -1),
        Wr3, br3.reshape(1, 1))
    return out.reshape(g)


# speedup vs baseline: 2.6319x; 1.3077x over previous
import functools

import jax
import jax.numpy as jnp
from jax import lax
from jax.experimental import pallas as pl
from jax.experimental.pallas import tpu as pltpu
from jax.experimental.pallas import tpu_sc as plsc

_NC = 2
_NS = 16
_LANES = 16


@functools.lru_cache(maxsize=None)
def _make_edge_kernel(n, e_edges, d):
    nw = _NC * _NS
    epw = e_edges // nw
    ch = 40
    assert epw % ch == 0
    nchunk = epw // ch
    assert nchunk % 2 == 0
    zr = 200
    assert n % zr == 0
    nzchunk = n // zr
    vpr = d // _LANES

    mesh = plsc.VectorSubcoreMesh(core_axis_name="c", subcore_axis_name="s")

    @functools.partial(
        pl.kernel,
        mesh=mesh,
        out_type=jax.ShapeDtypeStruct((_NC, n, d), jnp.float32),
        scratch_types=[
            pltpu.VMEM((ch,), jnp.int32),
            pltpu.VMEM((ch,), jnp.int32),
            pltpu.VMEM((ch,), jnp.int32),
            pltpu.VMEM((ch,), jnp.int32),
            pltpu.VMEM((ch, d), jnp.float32),
            pltpu.VMEM((ch, d), jnp.float32),
            pltpu.VMEM((ch, d), jnp.float32),
            pltpu.VMEM((ch, d), jnp.float32),
            pltpu.VMEM((zr, d), jnp.float32),
            pltpu.VMEM_SHARED((n, d), jnp.float32),
            pltpu.SemaphoreType.DMA,
            pltpu.SemaphoreType.DMA,
            pltpu.SemaphoreType.DMA,
            pltpu.SemaphoreType.DMA,
            pltpu.SemaphoreType.DMA,
            pltpu.SemaphoreType.DMA,
            pltpu.SemaphoreType.DMA,
            pltpu.SemaphoreType.DMA,
        ],
    )
    def edge_kernel(h_hbm, e_hbm, src_hbm, dst_hbm, out_hbm,
                    srcA, srcB, dstA, dstB, rowsA, rowsB, erowsA, erowsB,
                    zero_v, aggr_sh,
                    sem_ldA, sem_ldB, sem_eA, sem_eB,
                    sem_gA, sem_gB, sem_sA, sem_sB):
        c = lax.axis_index("c")
        s = lax.axis_index("s")
        bufs = ((srcA, dstA, rowsA, erowsA, sem_ldA, sem_eA, sem_gA, sem_sA),
                (srcB, dstB, rowsB, erowsB, sem_ldB, sem_eB, sem_gB, sem_sB))

        zeros16 = jnp.zeros((_LANES,), jnp.float32)

        def zbody(i, _):
            for q in range(vpr):
                zero_v[i, pl.ds(q * _LANES, _LANES)] = zeros16
            return 0

        lax.fori_loop(0, zr, zbody, 0)
        for j in range((nzchunk + _NS - 1) // _NS):
            k = s + j * _NS

            @pl.when(k < nzchunk)
            def _():
                pltpu.sync_copy(zero_v, aggr_sh.at[pl.ds(k * zr, zr)])

        plsc.subcore_barrier()

        base = (c * _NS + s) * epw

        def issue_loads(ci, b):
            src_v, dst_v, _, erows_v, sem_ld, sem_e, _, _ = bufs[b]
            off = base + ci * ch
            pltpu.async_copy(src_hbm.at[pl.ds(off, ch)], src_v, sem_ld)
            pltpu.async_copy(dst_hbm.at[pl.ds(off, ch)], dst_v, sem_ld)
            pltpu.async_copy(e_hbm.at[pl.ds(off, ch)], erows_v, sem_e)

        def wait_loads(ci, b):
            src_v, dst_v, _, _, sem_ld, _, _, _ = bufs[b]
            off = base + ci * ch
            pltpu.make_async_copy(src_hbm.at[pl.ds(off, ch)], src_v, sem_ld).wait()
            pltpu.make_async_copy(dst_hbm.at[pl.ds(off, ch)], dst_v, sem_ld).wait()

        def issue_gather(b):
            src_v, _, rows_v, _, _, _, sem_g, _ = bufs[b]
            pltpu.async_copy(h_hbm.at[src_v], rows_v, sem_g)

        def wait_prev_scatter(b):
            _, dst_v, rows_v, _, _, _, _, sem_s = bufs[b]
            pltpu.make_async_copy(rows_v, aggr_sh.at[dst_v], sem_s).wait()

        def step(c_cur, b_cur, c_nxt, prefetch, scatter_guard):
            src_v, dst_v, rows_v, erows_v, _, sem_e, sem_g, sem_s = bufs[b_cur]
            b_nxt = 1 - b_cur

            if prefetch:
                issue_loads(c_nxt, b_nxt)
            pltpu.make_async_copy(h_hbm.at[src_v], rows_v, sem_g).wait()
            off = base + c_cur * ch
            pltpu.make_async_copy(e_hbm.at[pl.ds(off, ch)], erows_v, sem_e).wait()

            def cbody(r, _):
                for q in range(vpr):
                    v = (rows_v[r, pl.ds(q * _LANES, _LANES)]
                         + erows_v[r, pl.ds(q * _LANES, _LANES)])
                    rows_v[r, pl.ds(q * _LANES, _LANES)] = jnp.maximum(v, 0.0)
                return 0

            lax.fori_loop(0, ch, cbody, 0)
            pltpu.async_copy(rows_v, aggr_sh.at[dst_v], sem_s, add=True)

            if prefetch:
                if scatter_guard is None:
                    wait_prev_scatter(b_nxt)
                else:
                    pl.when(scatter_guard)(lambda: wait_prev_scatter(b_nxt))
                wait_loads(c_nxt, b_nxt)
                issue_gather(b_nxt)

        issue_loads(0, 0)
        wait_loads(0, 0)
        issue_gather(0)

        def dbody(i, _):
            step(2 * i, 0, 2 * i + 1, True, i > 0)
            step(2 * i + 1, 1, 2 * i + 2, True, None)
            return 0

        lax.fori_loop(0, nchunk // 2 - 1, dbody, 0)
        step(nchunk - 2, 0, nchunk - 1, True, None)
        step(nchunk - 1, 1, nchunk, False, None)

        pltpu.make_async_copy(rowsA, aggr_sh.at[dstA], sem_sA).wait()
        pltpu.make_async_copy(rowsB, aggr_sh.at[dstB], sem_sB).wait()
        plsc.subcore_barrier()

        @pl.when(s == 0)
        def _():
            pltpu.sync_copy(aggr_sh, out_hbm.at[c])

    return edge_kernel


@functools.lru_cache(maxsize=None)
def _make_eproj(num_layers, e_edges, ed, d):
    eb = 3200
    assert e_edges % eb == 0
    grid = (num_layers, e_edges // eb)

    def body(ea_ref, we_ref, be_ref, out_ref):
        out_ref[0] = (jnp.dot(ea_ref[...], we_ref[0],
                              preferred_element_type=jnp.float32)
                      + be_ref[0])

    return pl.pallas_call(
        body,
        grid=grid,
        in_specs=[
            pl.BlockSpec((eb, ed), lambda l, i: (i, 0)),
            pl.BlockSpec((1, ed, d), lambda l, i: (l, 0, 0)),
            pl.BlockSpec((1, 1, d), lambda l, i: (l, 0, 0)),
        ],
        out_specs=pl.BlockSpec((1, eb, d), lambda l, i: (l, i, 0)),
        out_shape=jax.ShapeDtypeStruct((num_layers, e_edges, d), jnp.float32),
    )


@functools.lru_cache(maxsize=None)
def _make_update(n, d, h_dim):
    def body(h_ref, a_ref, w_ref, b_ref, g_ref, bt_ref, out_ref):
        t = h_ref[...] + a_ref[0] + a_ref[1]
        h2 = jnp.dot(t, w_ref[...], preferred_element_type=jnp.float32) + b_ref[...]
        mean = jnp.mean(h2, axis=0, keepdims=True)
        dvt = h2 - mean
        var = jnp.mean(dvt * dvt, axis=0, keepdims=True)
        out_ref[...] = jnp.maximum(
            g_ref[...] * dvt * lax.rsqrt(var + 1e-5) + bt_ref[...], 0.0)

    return pl.pallas_call(
        body,
        out_shape=jax.ShapeDtypeStruct((n, h_dim), jnp.float32),
    )


@functools.lru_cache(maxsize=None)
def _make_readout(n, h_dim, g):
    def body(h_ref, seg_ref, w1_ref, b1_ref, w2_ref, b2_ref, w3_ref, b3_ref,
             out_ref):
        onehot = jnp.where(
            seg_ref[...] == lax.broadcasted_iota(jnp.int32, (1, g), 1),
            1.0, 0.0)
        dn = (((0,), (0,)), ((), ()))
        pooled = lax.dot_general(onehot, h_ref[...], dn,
                                 preferred_element_type=jnp.float32)
        counts = lax.dot_general(onehot, jnp.ones((n, 1), jnp.float32), dn,
                                 preferred_element_type=jnp.float32)
        pooled = pooled / jnp.maximum(counts, 1.0)
        z = jnp.maximum(jnp.dot(pooled, w1_ref[...],
                                preferred_element_type=jnp.float32)
                        + b1_ref[...], 0.0)
        z = jnp.maximum(jnp.dot(z, w2_ref[...],
                                preferred_element_type=jnp.float32)
                        + b2_ref[...], 0.0)
        out_ref[...] = (jnp.dot(z, w3_ref[...],
                                preferred_element_type=jnp.float32)
                        + b3_ref[...])

    return pl.pallas_call(
        body,
        out_shape=jax.ShapeDtypeStruct((g, 1), jnp.float32),
    )


def kernel(x, edge_index, edge_attr, batch, W_nn, b_nn, W_edge, b_edge,
           bn_gamma, bn_beta, Wr1, br1, Wr2, br2, Wr3, br3):
    n, d = x.shape
    num_layers, ed, h_dim = W_edge.shape
    e_edges = edge_attr.shape[0]
    g = 64

    src = edge_index[0]
    dst = edge_index[1]

    eproj = _make_eproj(num_layers, e_edges, ed, d)(
        edge_attr, W_edge, b_edge.reshape(num_layers, 1, h_dim))

    edge_call = _make_edge_kernel(n, e_edges, d)
    upd_call = _make_update(n, d, h_dim)

    h = x
    for l in range(num_layers):
        aggr2 = edge_call(h, eproj[l], src, dst)
        h = upd_call(h, aggr2, W_nn[l],
                     b_nn[l].reshape(1, h_dim),
                     bn_gamma[l].reshape(1, h_dim),
                     bn_beta[l].reshape(1, h_dim))

    out = _make_readout(n, h_dim, g)(
        h, batch.reshape(n, 1),
        Wr1, br1.reshape(1, -1), Wr2, br2.reshape(1, -1),
        Wr3, br3.reshape(1, 1))
    return out.reshape(g)


# speedup vs baseline: 3.5889x; 1.3636x over previous
import functools

import jax
import jax.numpy as jnp
from jax import lax
from jax.experimental import pallas as pl
from jax.experimental.pallas import tpu as pltpu
from jax.experimental.pallas import tpu_sc as plsc

_NC = 2
_NS = 16
_LANES = 16


@functools.lru_cache(maxsize=None)
def _make_edge_kernel(n, e_edges, d):
    nw = _NC * _NS
    epw = e_edges // nw
    ch = 40
    assert epw % ch == 0
    nchunk = epw // ch
    assert nchunk % 2 == 0
    zr = 200
    assert n % zr == 0
    nzchunk = n // zr
    vpr = d // _LANES

    mesh = plsc.VectorSubcoreMesh(core_axis_name="c", subcore_axis_name="s")

    @functools.partial(
        pl.kernel,
        mesh=mesh,
        out_type=jax.ShapeDtypeStruct((_NC, n, d), jnp.float32),
        scratch_types=[
            pltpu.VMEM((ch,), jnp.int32),
            pltpu.VMEM((ch,), jnp.int32),
            pltpu.VMEM((ch,), jnp.int32),
            pltpu.VMEM((ch,), jnp.int32),
            pltpu.VMEM((ch, d), jnp.float32),
            pltpu.VMEM((ch, d), jnp.float32),
            pltpu.VMEM((ch, d), jnp.float32),
            pltpu.VMEM((ch, d), jnp.float32),
            pltpu.VMEM((zr, d), jnp.float32),
            pltpu.VMEM_SHARED((n, d), jnp.float32),
            pltpu.SemaphoreType.DMA,
            pltpu.SemaphoreType.DMA,
            pltpu.SemaphoreType.DMA,
            pltpu.SemaphoreType.DMA,
            pltpu.SemaphoreType.DMA,
            pltpu.SemaphoreType.DMA,
            pltpu.SemaphoreType.DMA,
            pltpu.SemaphoreType.DMA,
        ],
    )
    def edge_kernel(h_hbm, e_hbm, src_hbm, dst_hbm, out_hbm,
                    srcA, srcB, dstA, dstB, rowsA, rowsB, erowsA, erowsB,
                    zero_v, aggr_sh,
                    sem_ldA, sem_ldB, sem_eA, sem_eB,
                    sem_gA, sem_gB, sem_sA, sem_sB):
        c = lax.axis_index("c")
        s = lax.axis_index("s")
        bufs = ((srcA, dstA, rowsA, erowsA, sem_ldA, sem_eA, sem_gA, sem_sA),
                (srcB, dstB, rowsB, erowsB, sem_ldB, sem_eB, sem_gB, sem_sB))

        zeros16 = jnp.zeros((_LANES,), jnp.float32)

        def zbody(i, _):
            for q in range(vpr):
                zero_v[i, pl.ds(q * _LANES, _LANES)] = zeros16
            return 0

        lax.fori_loop(0, zr, zbody, 0)
        for j in range((nzchunk + _NS - 1) // _NS):
            k = s + j * _NS

            @pl.when(k < nzchunk)
            def _():
                pltpu.sync_copy(zero_v, aggr_sh.at[pl.ds(k * zr, zr)])

        plsc.subcore_barrier()

        base = (c * _NS + s) * epw

        def issue_loads(ci, b):
            src_v, dst_v, _, erows_v, sem_ld, sem_e, _, _ = bufs[b]
            off = base + ci * ch
            pltpu.async_copy(src_hbm.at[pl.ds(off, ch)], src_v, sem_ld)
            pltpu.async_copy(dst_hbm.at[pl.ds(off, ch)], dst_v, sem_ld)
            pltpu.async_copy(e_hbm.at[pl.ds(off, ch)], erows_v, sem_e)

        def wait_loads(ci, b):
            src_v, dst_v, _, _, sem_ld, _, _, _ = bufs[b]
            off = base + ci * ch
            pltpu.make_async_copy(src_hbm.at[pl.ds(off, ch)], src_v, sem_ld).wait()
            pltpu.make_async_copy(dst_hbm.at[pl.ds(off, ch)], dst_v, sem_ld).wait()

        def issue_gather(b):
            src_v, _, rows_v, _, _, _, sem_g, _ = bufs[b]
            pltpu.async_copy(h_hbm.at[src_v], rows_v, sem_g)

        def wait_prev_scatter(b):
            _, dst_v, rows_v, _, _, _, _, sem_s = bufs[b]
            pltpu.make_async_copy(rows_v, aggr_sh.at[dst_v], sem_s).wait()

        def step(c_cur, b_cur, c_nxt, prefetch, scatter_guard):
            src_v, dst_v, rows_v, erows_v, _, sem_e, sem_g, sem_s = bufs[b_cur]
            b_nxt = 1 - b_cur

            if prefetch:
                issue_loads(c_nxt, b_nxt)
            pltpu.make_async_copy(h_hbm.at[src_v], rows_v, sem_g).wait()
            off = base + c_cur * ch
            pltpu.make_async_copy(e_hbm.at[pl.ds(off, ch)], erows_v, sem_e).wait()

            def cbody(r, _):
                for q in range(vpr):
                    v = (rows_v[r, pl.ds(q * _LANES, _LANES)]
                         + erows_v[r, pl.ds(q * _LANES, _LANES)])
                    rows_v[r, pl.ds(q * _LANES, _LANES)] = jnp.maximum(v, 0.0)
                return 0

            lax.fori_loop(0, ch, cbody, 0)
            pltpu.async_copy(rows_v, aggr_sh.at[dst_v], sem_s, add=True)

            if prefetch:
                if scatter_guard is None:
                    wait_prev_scatter(b_nxt)
                else:
                    pl.when(scatter_guard)(lambda: wait_prev_scatter(b_nxt))
                wait_loads(c_nxt, b_nxt)
                issue_gather(b_nxt)

        issue_loads(0, 0)
        wait_loads(0, 0)
        issue_gather(0)

        def dbody(i, _):
            step(2 * i, 0, 2 * i + 1, True, i > 0)
            step(2 * i + 1, 1, 2 * i + 2, True, None)
            return 0

        lax.fori_loop(0, nchunk // 2 - 1, dbody, 0)
        step(nchunk - 2, 0, nchunk - 1, True, None)
        step(nchunk - 1, 1, nchunk, False, None)

        pltpu.make_async_copy(rowsA, aggr_sh.at[dstA], sem_sA).wait()
        pltpu.make_async_copy(rowsB, aggr_sh.at[dstB], sem_sB).wait()
        plsc.subcore_barrier()

        @pl.when(s == 0)
        def _():
            pltpu.sync_copy(aggr_sh, out_hbm.at[c])

    return edge_kernel


@functools.lru_cache(maxsize=None)
def _make_eproj(e_edges, ed, d):
    eb = 6400
    assert e_edges % eb == 0
    grid = (e_edges // eb,)

    def body(ea_ref, we_ref, be_ref, out_ref):
        out_ref[...] = (jnp.dot(ea_ref[...], we_ref[...],
                                preferred_element_type=jnp.float32)
                        + be_ref[...])

    return pl.pallas_call(
        body,
        grid=grid,
        in_specs=[
            pl.BlockSpec((eb, ed), lambda i: (i, 0)),
            pl.BlockSpec((ed, d), lambda i: (0, 0)),
            pl.BlockSpec((1, d), lambda i: (0, 0)),
        ],
        out_specs=pl.BlockSpec((eb, d), lambda i: (i, 0)),
        out_shape=jax.ShapeDtypeStruct((e_edges, d), jnp.float32),
    )


@functools.lru_cache(maxsize=None)
def _make_update(n, d, h_dim):
    def body(h_ref, a_ref, w_ref, b_ref, g_ref, bt_ref, out_ref):
        t = h_ref[...] + a_ref[0] + a_ref[1]
        h2 = jnp.dot(t, w_ref[...], preferred_element_type=jnp.float32) + b_ref[...]
        mean = jnp.mean(h2, axis=0, keepdims=True)
        dvt = h2 - mean
        var = jnp.mean(dvt * dvt, axis=0, keepdims=True)
        out_ref[...] = jnp.maximum(
            g_ref[...] * dvt * lax.rsqrt(var + 1e-5) + bt_ref[...], 0.0)

    return pl.pallas_call(
        body,
        out_shape=jax.ShapeDtypeStruct((n, h_dim), jnp.float32),
    )


@functools.lru_cache(maxsize=None)
def _make_readout(n, h_dim, g):
    def body(h_ref, seg_ref, w1_ref, b1_ref, w2_ref, b2_ref, w3_ref, b3_ref,
             out_ref):
        onehot = jnp.where(
            seg_ref[...] == lax.broadcasted_iota(jnp.int32, (1, g), 1),
            1.0, 0.0)
        dn = (((0,), (0,)), ((), ()))
        pooled = lax.dot_general(onehot, h_ref[...], dn,
                                 preferred_element_type=jnp.float32)
        counts = lax.dot_general(onehot, jnp.ones((n, 1), jnp.float32), dn,
                                 preferred_element_type=jnp.float32)
        pooled = pooled / jnp.maximum(counts, 1.0)
        z = jnp.maximum(jnp.dot(pooled, w1_ref[...],
                                preferred_element_type=jnp.float32)
                        + b1_ref[...], 0.0)
        z = jnp.maximum(jnp.dot(z, w2_ref[...],
                                preferred_element_type=jnp.float32)
                        + b2_ref[...], 0.0)
        out_ref[...] = (jnp.dot(z, w3_ref[...],
                                preferred_element_type=jnp.float32)
                        + b3_ref[...])

    return pl.pallas_call(
        body,
        out_shape=jax.ShapeDtypeStruct((g, 1), jnp.float32),
    )


def kernel(x, edge_index, edge_attr, batch, W_nn, b_nn, W_edge, b_edge,
           bn_gamma, bn_beta, Wr1, br1, Wr2, br2, Wr3, br3):
    n, d = x.shape
    num_layers, ed, h_dim = W_edge.shape
    e_edges = edge_attr.shape[0]
    g = 64

    src = edge_index[0]
    dst = edge_index[1]

    eproj_call = _make_eproj(e_edges, ed, d)
    eproj = [eproj_call(edge_attr, W_edge[l], b_edge[l].reshape(1, h_dim))
             for l in range(num_layers)]

    edge_call = _make_edge_kernel(n, e_edges, d)
    upd_call = _make_update(n, d, h_dim)

    h = x
    for l in range(num_layers):
        aggr2 = edge_call(h, eproj[l], src, dst)
        h = upd_call(h, aggr2, W_nn[l],
                     b_nn[l].reshape(1, h_dim),
                     bn_gamma[l].reshape(1, h_dim),
                     bn_beta[l].reshape(1, h_dim))

    out = _make_readout(n, h_dim, g)(
        h, batch.reshape(n, 1),
        Wr1, br1.reshape(1, -1), Wr2, br2.reshape(1, -1),
        Wr3, br3.reshape(1, 1))
    return out.reshape(g)


# speedup vs baseline: 4.2939x; 1.1965x over previous
import functools

import jax
import jax.numpy as jnp
from jax import lax
from jax.experimental import pallas as pl
from jax.experimental.pallas import tpu as pltpu
from jax.experimental.pallas import tpu_sc as plsc

_NC = 2
_NS = 16
_LANES = 16


@functools.lru_cache(maxsize=None)
def _make_edge_kernel(n, e_edges, d):
    nw = _NC * _NS
    epw = e_edges // nw
    ch = 40
    assert epw % ch == 0
    nchunk = epw // ch
    ri = 4
    rr = 3
    zr = ch
    assert n % zr == 0
    nzchunk = n // zr
    vpr = d // _LANES
    unroll = 12
    nsteady = (nchunk - 2 - 8) // unroll
    assert nchunk - 2 - unroll * nsteady == 8

    mesh = plsc.VectorSubcoreMesh(core_axis_name="c", subcore_axis_name="s")

    scratch = (
        [pltpu.VMEM((ch,), jnp.int32)] * ri +
        [pltpu.VMEM((ch,), jnp.int32)] * ri +
        [pltpu.VMEM((ch, d), jnp.float32)] * rr +
        [pltpu.VMEM((ch, d), jnp.float32)] * rr +
        [pltpu.VMEM_SHARED((n, d), jnp.float32)] +
        [pltpu.SemaphoreType.DMA] * ri +
        [pltpu.SemaphoreType.DMA] * (3 * rr)
    )

    @functools.partial(
        pl.kernel,
        mesh=mesh,
        out_type=jax.ShapeDtypeStruct((_NC, n, d), jnp.float32),
        scratch_types=scratch,
    )
    def edge_kernel(h_hbm, e_hbm, src_hbm, dst_hbm, out_hbm, *scr):
        srcs = scr[0:ri]
        dsts = scr[ri:2 * ri]
        rows = scr[2 * ri:2 * ri + rr]
        erows = scr[2 * ri + rr:2 * ri + 2 * rr]
        aggr_sh = scr[2 * ri + 2 * rr]
        p = 2 * ri + 2 * rr + 1
        sem_ld = scr[p:p + ri]
        sem_e = scr[p + ri:p + ri + rr]
        sem_g = scr[p + ri + rr:p + ri + 2 * rr]
        sem_s = scr[p + ri + 2 * rr:p + ri + 3 * rr]

        c = lax.axis_index("c")
        s = lax.axis_index("s")

        zeros16 = jnp.zeros((_LANES,), jnp.float32)

        def zbody(i, _):
            for q in range(vpr):
                rows[0][i, pl.ds(q * _LANES, _LANES)] = zeros16
            return 0

        lax.fori_loop(0, zr, zbody, 0)
        for j in range((nzchunk + _NS - 1) // _NS):
            k = s + j * _NS

            @pl.when(k < nzchunk)
            def _():
                pltpu.sync_copy(rows[0], aggr_sh.at[pl.ds(k * zr, zr)])

        plsc.subcore_barrier()

        base = (c * _NS + s) * epw

        def issue_loads(ci, si, sr):
            off = base + ci * ch
            pltpu.async_copy(src_hbm.at[pl.ds(off, ch)], srcs[si], sem_ld[si])
            pltpu.async_copy(dst_hbm.at[pl.ds(off, ch)], dsts[si], sem_ld[si])
            pltpu.async_copy(e_hbm.at[pl.ds(off, ch)], erows[sr], sem_e[sr])

        def wait_idx(ci, si):
            off = base + ci * ch
            pltpu.make_async_copy(src_hbm.at[pl.ds(off, ch)], srcs[si],
                                  sem_ld[si]).wait()
            pltpu.make_async_copy(dst_hbm.at[pl.ds(off, ch)], dsts[si],
                                  sem_ld[si]).wait()

        def wait_e(ci, sr):
            off = base + ci * ch
            pltpu.make_async_copy(e_hbm.at[pl.ds(off, ch)], erows[sr],
                                  sem_e[sr]).wait()

        def issue_gather(si, sr):
            pltpu.async_copy(h_hbm.at[srcs[si]], rows[sr], sem_g[sr])

        def wait_gather(si, sr):
            pltpu.make_async_copy(h_hbm.at[srcs[si]], rows[sr],
                                  sem_g[sr]).wait()

        def issue_scatter(si, sr):
            pltpu.async_copy(rows[sr], aggr_sh.at[dsts[si]], sem_s[sr],
                             add=True)

        def wait_scatter(si, sr):
            pltpu.make_async_copy(rows[sr], aggr_sh.at[dsts[si]],
                                  sem_s[sr]).wait()

        def compute(sr):
            rv = rows[sr]
            ev = erows[sr]

            def cbody(r, _):
                for q in range(vpr):
                    v = (rv[r, pl.ds(q * _LANES, _LANES)]
                         + ev[r, pl.ds(q * _LANES, _LANES)])
                    rv[r, pl.ds(q * _LANES, _LANES)] = jnp.maximum(v, 0.0)
                return 0

            lax.fori_loop(0, ch, cbody, 0)

        def step(cc, k, wait_sc, pre2, pre1):
            si, sr = k % ri, k % rr
            if wait_sc:
                wait_gather(si, sr)
                wait_e(cc, sr)
                wait_scatter((k - 2) % ri, (k - 2) % rr)
            else:
                wait_gather(si, sr)
                wait_e(cc, sr)
            if pre2:
                issue_loads(cc + 2, (k + 2) % ri, (k + 2) % rr)
            if pre1:
                wait_idx(cc + 1, (k + 1) % ri)
                issue_gather((k + 1) % ri, (k + 1) % rr)
            compute(sr)
            issue_scatter(si, sr)

        issue_loads(0, 0, 0)
        issue_loads(1, 1, 1)
        wait_idx(0, 0)
        issue_gather(0, 0)
        step(0, 0, False, True, True)
        step(1, 1, False, True, True)

        def qbody(i, _):
            cc = 2 + unroll * i
            for k in range(unroll):
                step(cc + k, 2 + k, True, True, True)
            return 0

        lax.fori_loop(0, nsteady, qbody, 0)

        for cc in range(2 + unroll * nsteady, nchunk):
            step(cc, cc, True, cc + 2 < nchunk, cc + 1 < nchunk)
        wait_scatter((nchunk - 2) % ri, (nchunk - 2) % rr)
        wait_scatter((nchunk - 1) % ri, (nchunk - 1) % rr)
        plsc.subcore_barrier()

        for j in range((nzchunk + _NS - 1) // _NS):
            k = s + j * _NS

            @pl.when(k < nzchunk)
            def _():
                pltpu.sync_copy(aggr_sh.at[pl.ds(k * zr, zr)],
                                out_hbm.at[c, pl.ds(k * zr, zr)])

    return edge_kernel


@functools.lru_cache(maxsize=None)
def _make_eproj(e_edges, ed, d):
    eb = 6400
    assert e_edges % eb == 0
    grid = (e_edges // eb,)

    def body(ea_ref, we_ref, be_ref, out_ref):
        out_ref[...] = (jnp.dot(ea_ref[...], we_ref[...],
                                preferred_element_type=jnp.float32)
                        + be_ref[...])

    return pl.pallas_call(
        body,
        grid=grid,
        in_specs=[
            pl.BlockSpec((eb, ed), lambda i: (i, 0)),
            pl.BlockSpec((ed, d), lambda i: (0, 0)),
            pl.BlockSpec((1, d), lambda i: (0, 0)),
        ],
        out_specs=pl.BlockSpec((eb, d), lambda i: (i, 0)),
        out_shape=jax.ShapeDtypeStruct((e_edges, d), jnp.float32),
    )


@functools.lru_cache(maxsize=None)
def _make_update(n, d, h_dim):
    def body(h_ref, a_ref, w_ref, b_ref, g_ref, bt_ref, out_ref):
        t = h_ref[...] + a_ref[0] + a_ref[1]
        h2 = jnp.dot(t, w_ref[...], preferred_element_type=jnp.float32) + b_ref[...]
        mean = jnp.mean(h2, axis=0, keepdims=True)
        dvt = h2 - mean
        var = jnp.mean(dvt * dvt, axis=0, keepdims=True)
        out_ref[...] = jnp.maximum(
            g_ref[...] * dvt * lax.rsqrt(var + 1e-5) + bt_ref[...], 0.0)

    return pl.pallas_call(
        body,
        out_shape=jax.ShapeDtypeStruct((n, h_dim), jnp.float32),
    )


@functools.lru_cache(maxsize=None)
def _make_readout(n, h_dim, g):
    def body(h_ref, seg_ref, w1_ref, b1_ref, w2_ref, b2_ref, w3_ref, b3_ref,
             out_ref):
        onehot = jnp.where(
            seg_ref[...] == lax.broadcasted_iota(jnp.int32, (1, g), 1),
            1.0, 0.0)
        dn = (((0,), (0,)), ((), ()))
        pooled = lax.dot_general(onehot, h_ref[...], dn,
                                 preferred_element_type=jnp.float32)
        counts = lax.dot_general(onehot, jnp.ones((n, 1), jnp.float32), dn,
                                 preferred_element_type=jnp.float32)
        pooled = pooled / jnp.maximum(counts, 1.0)
        z = jnp.maximum(jnp.dot(pooled, w1_ref[...],
                                preferred_element_type=jnp.float32)
                        + b1_ref[...], 0.0)
        z = jnp.maximum(jnp.dot(z, w2_ref[...],
                                preferred_element_type=jnp.float32)
                        + b2_ref[...], 0.0)
        out_ref[...] = (jnp.dot(z, w3_ref[...],
                                preferred_element_type=jnp.float32)
                        + b3_ref[...])

    return pl.pallas_call(
        body,
        out_shape=jax.ShapeDtypeStruct((g, 1), jnp.float32),
    )


def kernel(x, edge_index, edge_attr, batch, W_nn, b_nn, W_edge, b_edge,
           bn_gamma, bn_beta, Wr1, br1, Wr2, br2, Wr3, br3):
    n, d = x.shape
    num_layers, ed, h_dim = W_edge.shape
    e_edges = edge_attr.shape[0]
    g = 64

    src = edge_index[0]
    dst = edge_index[1]

    eproj_call = _make_eproj(e_edges, ed, d)
    eproj = [eproj_call(edge_attr, W_edge[l], b_edge[l].reshape(1, h_dim))
             for l in range(num_layers)]

    edge_call = _make_edge_kernel(n, e_edges, d)
    upd_call = _make_update(n, d, h_dim)

    h = x
    for l in range(num_layers):
        aggr2 = edge_call(h, eproj[l], src, dst)
        h = upd_call(h, aggr2, W_nn[l],
                     b_nn[l].reshape(1, h_dim),
                     bn_gamma[l].reshape(1, h_dim),
                     bn_beta[l].reshape(1, h_dim))

    out = _make_readout(n, h_dim, g)(
        h, batch.reshape(n, 1),
        Wr1, br1.reshape(1, -1), Wr2, br2.reshape(1, -1),
        Wr3, br3.reshape(1, 1))
    return out.reshape(g)
